# baseline (speedup 1.0000x reference)
import functools

import jax
import jax.numpy as jnp
from jax import lax
from jax.experimental import pallas as pl
from jax.experimental.pallas import tpu as pltpu
from jax.experimental.pallas import tpu_sc as plsc

NC = 2
NS = 16
NW = NC * NS

D = 64
BLK = 128
NBUF = 4
H = 200
B = 4096


def _sc_gather_kernel():
    mesh = plsc.VectorSubcoreMesh(core_axis_name="c", subcore_axis_name="s")

    @functools.partial(
        pl.kernel,
        mesh=mesh,
        out_type=jax.ShapeDtypeStruct((H, 8, NW, 8, BLK), jnp.float32),
        compiler_params=pltpu.CompilerParams(use_tc_tiling_on_sc=False,
                                             needs_layout_passes=False),
        scratch_types=[
            pltpu.VMEM((H // 8, 8, BLK), jnp.int32),
            pltpu.VMEM((NBUF, BLK, D), jnp.float32),
            pltpu.VMEM((NBUF, 8, 8, BLK), jnp.float32),
            pltpu.SemaphoreType.DMA((NBUF,)),
            pltpu.SemaphoreType.DMA((NBUF,)),
        ],
    )
    def k(table_hbm, x4_hbm, out_hbm, idx_v, g_v, t_v, sem_g, sem_o):
        wid = lax.axis_index("s") * NC + lax.axis_index("c")

        pltpu.sync_copy(x4_hbm.at[:, wid], idx_v)

        iota = lax.iota(jnp.int32, 16)
        bidx = [iota + (kk * 16) for kk in range(8)]

        def idx_slice(h):
            return idx_v.at[h // 8, h % 8]

        def start_gather(h, s):
            pltpu.async_copy(table_hbm.at[idx_slice(h)], g_v.at[s],
                             sem_g.at[s])

        def wait_gather(h, s):
            pltpu.make_async_copy(table_hbm.at[idx_slice(h)], g_v.at[s],
                                  sem_g.at[s]).wait()

        def start_write(h, s):
            pltpu.async_copy(t_v.at[s], out_hbm.at[h, :, wid], sem_o.at[s])

        def wait_write(h, s):
            pltpu.make_async_copy(t_v.at[s], out_hbm.at[h, :, wid],
                                  sem_o.at[s]).wait()

        def transpose(s):
            g_s = g_v.at[s]

            def tr_body(d, _):
                dvec = jnp.full((16,), 0, jnp.int32) + d
                d_hi = d // 8
                d_lo = d % 8
                for kk in range(8):
                    vals = plsc.load_gather(g_s, [bidx[kk], dvec])
                    t_v[s, d_hi, d_lo, pl.ds(kk * 16, 16)] = vals
                return _

            lax.fori_loop(0, D, tr_body, 0, unroll=False)

        def step(h, s, first):
            if not first:
                wait_write(h, s)
            wait_gather(h, s)
            transpose(s)
            start_write(h, s)
            start_gather(h + NBUF, s)

        for h in range(NBUF):
            start_gather(h, h)
        for h in range(NBUF):
            step(h, h, first=True)

        def body(g, _):
            base = g * NBUF
            for s in range(NBUF):
                step(base + s, s, first=False)
            return _

        lax.fori_loop(1, H // NBUF - 1, body, 0, unroll=False)

        for hh in range(H - NBUF, H):
            s = hh % NBUF
            wait_write(hh, s)
            wait_gather(hh, s)
            transpose(s)
            start_write(hh, s)
        for hh in range(H - NBUF, H):
            wait_write(hh, hh % NBUF)

    return k


@jax.jit
def kernel(x, table):
    x4 = x.astype(jnp.int32).reshape(NW, BLK, H // 8, 8).transpose(2, 0, 3, 1)
    out5 = _sc_gather_kernel()(table, x4)
    return out5.transpose(2, 4, 0, 1, 3).reshape(B, H, D)
